# Initial kernel scaffold; baseline (speedup 1.0000x reference)
import jax
import jax.numpy as jnp
from jax.experimental import pallas as pl


def kernel(x_batch, neighbor_id_lstlst, weights, bias):
    raise NotImplementedError("write your pallas kernel here")


# speedup vs baseline: 1.0258x; 1.0258x over previous
import jax
import jax.numpy as jnp
from jax.experimental import pallas as pl


def _conv_body(g_ref, w_ref, bias_ref, out_ref):
    gb = g_ref[...]
    wb = w_ref[...]
    acc = jax.lax.dot_general(
        gb, wb,
        dimension_numbers=(((2,), (1,)), ((1,), (0,))),
        preferred_element_type=jnp.float32)
    acc = acc + bias_ref[...][None, None, :]
    out_ref[...] = jnp.where(acc > 0, acc, jnp.exp(jnp.minimum(acc, 0.0)) - 1.0)


def kernel(x_batch, neighbor_id_lstlst, weights, bias):
    B, N, CIN = x_batch.shape
    M = neighbor_id_lstlst.shape[1]
    COUT = weights.shape[2]
    K = M * CIN
    g = jnp.take(x_batch, neighbor_id_lstlst.reshape(-1), axis=1)
    g = g.reshape(B, N, K)
    wk = weights.transpose(0, 1, 3, 2).reshape(N, K, COUT)
    P = 400
    out = pl.pallas_call(
        _conv_body,
        grid=(N // P,),
        in_specs=[
            pl.BlockSpec((B, P, K), lambda i: (0, i, 0)),
            pl.BlockSpec((P, K, COUT), lambda i: (i, 0, 0)),
            pl.BlockSpec((COUT,), lambda i: (0,)),
        ],
        out_specs=pl.BlockSpec((P, B, COUT), lambda i: (i, 0, 0)),
        out_shape=jax.ShapeDtypeStruct((N, B, COUT), jnp.float32),
    )(g, wk, bias)
    return out.transpose(1, 0, 2)
